# Initial kernel scaffold; baseline (speedup 1.0000x reference)
import jax
import jax.numpy as jnp
from jax.experimental import pallas as pl


def kernel(doc_features, word_features, edge_index, mode, lin_W, lin_b, W1, b1, W2, b2):
    raise NotImplementedError("write your pallas kernel here")


# speedup vs baseline: 11.2123x; 11.2123x over previous
import functools

import jax
import jax.numpy as jnp
from jax import lax
from jax.experimental import pallas as pl
from jax.experimental.pallas import tpu as pltpu
from jax.experimental.pallas import tpu_sc as plsc

N_DOC = 2000
N_NODES = 10000
N_EDGES = 320000
D = 128
NPAD = 10240

NC, NS = 2, 16
NW = NC * NS
EPT = N_EDGES // NW
K = 80
NSTEPS = EPT // K
RPT = NPAD // NS

@functools.cache
def _sc_kernels():
    mesh = plsc.VectorSubcoreMesh(core_axis_name="c", subcore_axis_name="s",
                                  num_cores=NC, num_subcores=NS)
    deg = functools.partial(
        pl.kernel,
        out_type=jax.ShapeDtypeStruct((NC * NPAD, D), jnp.float32),
        mesh=mesh,
        scratch_types=[
            pltpu.VMEM((K,), jnp.int32),
            pltpu.VMEM((K, D), jnp.float32),
            pltpu.VMEM_SHARED((NPAD, D), jnp.float32),
        ],
    )(_deg_body)
    scat = functools.partial(
        pl.kernel,
        out_type=jax.ShapeDtypeStruct((NC * NPAD, D), jnp.float32),
        mesh=mesh,
        scratch_types=[
            pltpu.VMEM((K,), jnp.int32),
            pltpu.VMEM((K,), jnp.int32),
            pltpu.VMEM((K, D), jnp.float32),
            pltpu.VMEM_SHARED((NPAD, D), jnp.float32),
            pltpu.SemaphoreType.DMA,
        ],
    )(_edge_scatter_body)
    return deg, scat


def _deg_body(dst_hbm, onerows_hbm, zerosd_hbm, out_hbm, didx_v, ones_v, acc_s):
    c = lax.axis_index("c")
    s = lax.axis_index("s")
    base = c * (NS * EPT) + s * EPT
    r0 = s * RPT
    pltpu.sync_copy(zerosd_hbm.at[pl.ds(r0, RPT)], acc_s.at[pl.ds(r0, RPT)])
    pltpu.sync_copy(onerows_hbm, ones_v)
    plsc.subcore_barrier()

    @pl.loop(0, NSTEPS)
    def _(j):
        pltpu.sync_copy(dst_hbm.at[pl.ds(base + j * K, K)], didx_v)
        pltpu.sync_copy(ones_v, acc_s.at[didx_v], add=True)

    plsc.subcore_barrier()
    pltpu.sync_copy(acc_s.at[pl.ds(r0, RPT)],
                    out_hbm.at[pl.ds(c * NPAD + r0, RPT)])


def _edge_scatter_body(src_hbm, dst_hbm, g_hbm, zerosd_hbm, out_hbm,
                       sidx_v, didx_v, rows_v, acc_s, sem):
    c = lax.axis_index("c")
    s = lax.axis_index("s")
    base = c * (NS * EPT) + s * EPT
    r0 = s * RPT
    pltpu.sync_copy(zerosd_hbm.at[pl.ds(r0, RPT)], acc_s.at[pl.ds(r0, RPT)])
    plsc.subcore_barrier()

    @pl.loop(0, NSTEPS)
    def _(j):
        e0 = base + j * K
        pltpu.sync_copy(src_hbm.at[pl.ds(e0, K)], sidx_v)
        pltpu.sync_copy(dst_hbm.at[pl.ds(e0, K)], didx_v)
        pltpu.async_copy(g_hbm.at[sidx_v], rows_v, sem).wait()
        pltpu.sync_copy(rows_v, acc_s.at[didx_v], add=True)

    plsc.subcore_barrier()
    pltpu.sync_copy(acc_s.at[pl.ds(r0, RPT)],
                    out_hbm.at[pl.ds(c * NPAD + r0, RPT)])


def _dinv_from_hist(hist):
    deg = hist[0:N_NODES, 0:1] + hist[NPAD:NPAD + N_NODES, 0:1] + 1.0
    return lax.rsqrt(deg)


def _tc1_body(doc_ref, word_ref, linw_ref, linb_ref, w1_ref, hist_ref, g1_ref):
    dinv = _dinv_from_hist(hist_ref[...])
    wf = jnp.dot(word_ref[...], linw_ref[...],
                 preferred_element_type=jnp.float32) + linb_ref[...]
    hd = jnp.dot(doc_ref[...], w1_ref[...], preferred_element_type=jnp.float32)
    hw = jnp.dot(wf, w1_ref[...], preferred_element_type=jnp.float32)
    g1_ref[0:N_DOC, :] = dinv[0:N_DOC] * hd
    g1_ref[N_DOC:N_NODES, :] = dinv[N_DOC:N_NODES] * hw


def _tc2_body(hist_ref, acc_ref, g1_ref, b1_ref, w2_ref, g2_ref):
    dinv = _dinv_from_hist(hist_ref[...])
    t = acc_ref[0:N_NODES, :] + acc_ref[NPAD:NPAD + N_NODES, :] + g1_ref[...]
    z = jnp.maximum(dinv * t + b1_ref[...], 0.0)
    g2_ref[...] = dinv * jnp.dot(z, w2_ref[...],
                                 preferred_element_type=jnp.float32)


def _tc3_body(hist_ref, acc_ref, g2_ref, b2_ref, out_ref):
    dinv = _dinv_from_hist(hist_ref[...])
    t = acc_ref[0:N_NODES, :] + acc_ref[NPAD:NPAD + N_NODES, :] + g2_ref[...]
    out_ref[...] = dinv * t + b2_ref[...]


def _vmem_call(body, n_in, out_shape):
    return pl.pallas_call(
        body,
        out_shape=out_shape,
        in_specs=[pl.BlockSpec(memory_space=pltpu.VMEM)] * n_in,
        out_specs=pl.BlockSpec(memory_space=pltpu.VMEM),
    )


def kernel(doc_features, word_features, edge_index, mode,
           lin_W, lin_b, W1, b1, W2, b2):
    src = edge_index[0].astype(jnp.int32)
    dst = edge_index[1].astype(jnp.int32)

    onerows = jnp.zeros((K, D), jnp.float32).at[:, 0].set(1.0)
    zerosd = jnp.zeros((NPAD, D), jnp.float32)

    deg_k, scat_k = _sc_kernels()
    hist = deg_k(dst, onerows, zerosd)

    g1 = _vmem_call(_tc1_body, 6,
                    jax.ShapeDtypeStruct((N_NODES, D), jnp.float32))(
        doc_features, word_features, lin_W, lin_b.reshape(1, D), W1, hist)

    acc1 = scat_k(src, dst, g1, zerosd)

    g2 = _vmem_call(_tc2_body, 5,
                    jax.ShapeDtypeStruct((N_NODES, D), jnp.float32))(
        hist, acc1, g1, b1.reshape(1, D), W2)

    acc2 = scat_k(src, dst, g2, zerosd)

    out = _vmem_call(_tc3_body, 4,
                     jax.ShapeDtypeStruct((N_NODES, D), jnp.float32))(
        hist, acc2, g2, b2.reshape(1, D))
    return out


# speedup vs baseline: 23.2397x; 2.0727x over previous
import functools

import jax
import jax.numpy as jnp
from jax import lax
from jax.experimental import pallas as pl
from jax.experimental.pallas import tpu as pltpu
from jax.experimental.pallas import tpu_sc as plsc

N_DOC = 2000
N_NODES = 10000
N_EDGES = 320000
D = 128
NPAD = 10240

NC, NS = 2, 16
NW = NC * NS
EPT = N_EDGES // NW
K = 80
NSTEPS = EPT // K
RPT = NPAD // NS
L = 16
CH = 2000


@functools.cache
def _sc_kernels():
    mesh = plsc.VectorSubcoreMesh(core_axis_name="c", subcore_axis_name="s",
                                  num_cores=NC, num_subcores=NS)
    deg = functools.partial(
        pl.kernel,
        out_type=jax.ShapeDtypeStruct((NW, NPAD), jnp.float32),
        mesh=mesh,
        compiler_params=pltpu.CompilerParams(needs_layout_passes=False),
        scratch_types=[
            pltpu.VMEM((CH,), jnp.int32),
            pltpu.VMEM((NPAD,), jnp.float32),
        ],
    )(_deg_body)
    scat = functools.partial(
        pl.kernel,
        out_type=jax.ShapeDtypeStruct((NC * NPAD, D), jnp.float32),
        mesh=mesh,
        scratch_types=[
            pltpu.VMEM((EPT,), jnp.int32),
            pltpu.VMEM((EPT,), jnp.int32),
            pltpu.VMEM((K,), jnp.int32),
            pltpu.VMEM((K,), jnp.int32),
            pltpu.VMEM((K, D), jnp.float32),
            pltpu.VMEM((K, D), jnp.float32),
            pltpu.VMEM_SHARED((NPAD, D), jnp.float32),
            pltpu.SemaphoreType.DMA,
            pltpu.SemaphoreType.DMA,
            pltpu.SemaphoreType.DMA,
            pltpu.SemaphoreType.DMA,
        ],
    )(_edge_scatter_body)
    return deg, scat


def _deg_body(dst_hbm, out_hbm, didx_v, hist_v):
    c = lax.axis_index("c")
    s = lax.axis_index("s")
    wid = c * NS + s
    base = wid * EPT
    zero = jnp.zeros((L,), jnp.float32)

    @pl.loop(0, NPAD // L)
    def _(i):
        hist_v[pl.ds(i * L, L)] = zero

    ones = jnp.ones((L,), jnp.float32)

    @pl.loop(0, EPT // CH)
    def _(jc):
        pltpu.sync_copy(dst_hbm.at[pl.ds(base + jc * CH, CH)], didx_v)

        @pl.loop(0, CH // L)
        def _(i):
            idx = didx_v[pl.ds(i * L, L)]
            plsc.addupdate_scatter(hist_v, [idx], ones)

    pltpu.sync_copy(hist_v, out_hbm.at[wid])


def _edge_scatter_body(src_hbm, dst_hbm, g_hbm, zerosd_hbm, out_hbm,
                       srcall_v, dstall_v, didx0_v, didx1_v, rows0_v, rows1_v,
                       acc_s, gsem0, gsem1, ssem0, ssem1):
    c = lax.axis_index("c")
    s = lax.axis_index("s")
    base = (c * NS + s) * EPT
    r0 = s * RPT
    didx = (didx0_v, didx1_v)
    rows = (rows0_v, rows1_v)
    gsem = (gsem0, gsem1)
    ssem = (ssem0, ssem1)

    pltpu.sync_copy(zerosd_hbm.at[pl.ds(r0, RPT)], acc_s.at[pl.ds(r0, RPT)])
    pltpu.sync_copy(src_hbm.at[pl.ds(base, EPT)], srcall_v)
    pltpu.sync_copy(dst_hbm.at[pl.ds(base, EPT)], dstall_v)
    plsc.subcore_barrier()

    def stage_didx(j, b):
        for i in range(K // L):
            didx[b][pl.ds(i * L, L)] = dstall_v[pl.ds(j * K + i * L, L)]

    def fire_gather(j, b):
        stage_didx(j, b)
        return pltpu.async_copy(g_hbm.at[srcall_v.at[pl.ds(j * K, K)]],
                                rows[b], gsem[b])

    def fire_scatter(b):
        return pltpu.async_copy(rows[b], acc_s.at[didx[b]], ssem[b],
                                add=True)

    @pl.loop(0, NSTEPS // 2)
    def _(p):
        j = 2 * p
        gd0 = fire_gather(j, 0)
        gd1 = fire_gather(j + 1, 1)
        gd0.wait()
        sd0 = fire_scatter(0)
        gd1.wait()
        sd1 = fire_scatter(1)
        sd0.wait()
        sd1.wait()

    gd = fire_gather(NSTEPS - 1, 0)
    gd.wait()
    fire_scatter(0).wait()

    plsc.subcore_barrier()
    pltpu.sync_copy(acc_s.at[pl.ds(r0, RPT)],
                    out_hbm.at[pl.ds(c * NPAD + r0, RPT)])


def _dinv_from_hist(hist):
    deg = jnp.sum(hist, axis=0)[0:N_NODES, None] + 1.0
    return lax.rsqrt(deg)


def _tc1_body(doc_ref, word_ref, linw_ref, linb_ref, w1_ref, hist_ref, g1_ref):
    dinv = _dinv_from_hist(hist_ref[...])
    wf = jnp.dot(word_ref[...], linw_ref[...],
                 preferred_element_type=jnp.float32) + linb_ref[...]
    hd = jnp.dot(doc_ref[...], w1_ref[...], preferred_element_type=jnp.float32)
    hw = jnp.dot(wf, w1_ref[...], preferred_element_type=jnp.float32)
    g1_ref[0:N_DOC, :] = dinv[0:N_DOC] * hd
    g1_ref[N_DOC:N_NODES, :] = dinv[N_DOC:N_NODES] * hw


def _tc2_body(hist_ref, acc_ref, g1_ref, b1_ref, w2_ref, g2_ref):
    dinv = _dinv_from_hist(hist_ref[...])
    t = acc_ref[0:N_NODES, :] + acc_ref[NPAD:NPAD + N_NODES, :] + g1_ref[...]
    z = jnp.maximum(dinv * t + b1_ref[...], 0.0)
    g2_ref[...] = dinv * jnp.dot(z, w2_ref[...],
                                 preferred_element_type=jnp.float32)


def _tc3_body(hist_ref, acc_ref, g2_ref, b2_ref, out_ref):
    dinv = _dinv_from_hist(hist_ref[...])
    t = acc_ref[0:N_NODES, :] + acc_ref[NPAD:NPAD + N_NODES, :] + g2_ref[...]
    out_ref[...] = dinv * t + b2_ref[...]


def _vmem_call(body, n_in, out_shape):
    return pl.pallas_call(
        body,
        out_shape=out_shape,
        in_specs=[pl.BlockSpec(memory_space=pltpu.VMEM)] * n_in,
        out_specs=pl.BlockSpec(memory_space=pltpu.VMEM),
    )


def kernel(doc_features, word_features, edge_index, mode,
           lin_W, lin_b, W1, b1, W2, b2):
    src = edge_index[0].astype(jnp.int32)
    dst = edge_index[1].astype(jnp.int32)

    zerosd = jnp.zeros((NPAD, D), jnp.float32)

    deg_k, scat_k = _sc_kernels()
    hist = deg_k(dst)

    g1 = _vmem_call(_tc1_body, 6,
                    jax.ShapeDtypeStruct((N_NODES, D), jnp.float32))(
        doc_features, word_features, lin_W, lin_b.reshape(1, D), W1, hist)

    acc1 = scat_k(src, dst, g1, zerosd)

    g2 = _vmem_call(_tc2_body, 5,
                    jax.ShapeDtypeStruct((N_NODES, D), jnp.float32))(
        hist, acc1, g1, b1.reshape(1, D), W2)

    acc2 = scat_k(src, dst, g2, zerosd)

    out = _vmem_call(_tc3_body, 4,
                     jax.ShapeDtypeStruct((N_NODES, D), jnp.float32))(
        hist, acc2, g2, b2.reshape(1, D))
    return out


# speedup vs baseline: 28.9426x; 1.2454x over previous
import functools

import jax
import jax.numpy as jnp
from jax import lax
from jax.experimental import pallas as pl
from jax.experimental.pallas import tpu as pltpu
from jax.experimental.pallas import tpu_sc as plsc

N_DOC = 2000
N_NODES = 10000
N_EDGES = 320000
D = 128
NPAD = 10240

NC, NS = 2, 16
NW = NC * NS
EPT = N_EDGES // NW
K = 80
NSTEPS = EPT // K
RPT = NPAD // NS
L = 16
CH = 2000
NBUF = 4


@functools.cache
def _sc_kernels():
    mesh = plsc.VectorSubcoreMesh(core_axis_name="c", subcore_axis_name="s",
                                  num_cores=NC, num_subcores=NS)
    deg = functools.partial(
        pl.kernel,
        out_type=jax.ShapeDtypeStruct((NW, NPAD), jnp.float32),
        mesh=mesh,
        compiler_params=pltpu.CompilerParams(needs_layout_passes=False),
        scratch_types=[
            pltpu.VMEM((CH,), jnp.int32),
            pltpu.VMEM((NPAD,), jnp.float32),
        ],
    )(_deg_body)
    scat = functools.partial(
        pl.kernel,
        out_type=jax.ShapeDtypeStruct((NC * NPAD, D), jnp.float32),
        mesh=mesh,
        scratch_types=(
            [pltpu.VMEM((EPT,), jnp.int32),
             pltpu.VMEM((EPT,), jnp.int32)]
            + [pltpu.VMEM((K,), jnp.int32)] * 2
            + [pltpu.VMEM((K, D), jnp.float32)] * 2
            + [pltpu.VMEM_SHARED((NPAD, D), jnp.float32)]
            + [pltpu.SemaphoreType.DMA] * 4
        ),
    )(_edge_scatter_body)
    return deg, scat


def _deg_body(dst_hbm, out_hbm, didx_v, hist_v):
    c = lax.axis_index("c")
    s = lax.axis_index("s")
    wid = c * NS + s
    base = wid * EPT
    zero = jnp.zeros((L,), jnp.float32)

    @pl.loop(0, NPAD // L)
    def _(i):
        hist_v[pl.ds(i * L, L)] = zero

    ones = jnp.ones((L,), jnp.float32)

    @pl.loop(0, EPT // CH)
    def _(jc):
        pltpu.sync_copy(dst_hbm.at[pl.ds(base + jc * CH, CH)], didx_v)

        @pl.loop(0, CH // L)
        def _(i):
            idx = didx_v[pl.ds(i * L, L)]
            plsc.addupdate_scatter(hist_v, [idx], ones)

    pltpu.sync_copy(hist_v, out_hbm.at[wid])


def _edge_scatter_body(src_hbm, dst_hbm, g_hbm, zerosd_hbm, out_hbm,
                       srcall_v, dstall_v, didx0_v, didx1_v, rows0_v, rows1_v,
                       acc_s, gsem0, gsem1, ssem0, ssem1):
    didx = (didx0_v, didx1_v)
    rows = (rows0_v, rows1_v)
    gsem = (gsem0, gsem1)
    ssem = (ssem0, ssem1)
    c = lax.axis_index("c")
    s = lax.axis_index("s")
    base = (c * NS + s) * EPT
    r0 = s * RPT

    pltpu.sync_copy(zerosd_hbm.at[pl.ds(r0, RPT)], acc_s.at[pl.ds(r0, RPT)])
    pltpu.sync_copy(src_hbm.at[pl.ds(base, EPT)], srcall_v)
    pltpu.sync_copy(dst_hbm.at[pl.ds(base, EPT)], dstall_v)
    plsc.subcore_barrier()

    def stage_didx(j, b):
        for i in range(K // L):
            didx[b][pl.ds(i * L, L)] = dstall_v[pl.ds(j * K + i * L, L)]

    def gfire(j, b):
        pltpu.async_copy(g_hbm.at[srcall_v.at[pl.ds(j * K, K)]],
                         rows[b], gsem[b])

    def gwait(b):
        pltpu.make_async_copy(g_hbm.at[srcall_v.at[pl.ds(0, K)]],
                              rows[b], gsem[b]).wait()

    def sfire(b):
        pltpu.async_copy(rows[b], acc_s.at[didx[b]], ssem[b], add=True)

    def swait(b):
        pltpu.make_async_copy(rows[b], acc_s.at[didx[b]], ssem[b]).wait()

    def body(j, b, first=False):
        bn = 1 - b
        if not first:
            swait(bn)
        stage_didx(j + 1, bn)
        gfire(j + 1, bn)
        gwait(b)
        sfire(b)

    stage_didx(0, 0)
    gfire(0, 0)
    body(0, 0, first=True)
    body(1, 1)

    @pl.loop(0, (NSTEPS - 3) // 2)
    def _(p):
        j = 2 + 2 * p
        body(j, 0)
        body(j + 1, 1)

    swait(1)
    gwait(0)
    sfire(0)
    swait(0)

    plsc.subcore_barrier()
    pltpu.sync_copy(acc_s.at[pl.ds(r0, RPT)],
                    out_hbm.at[pl.ds(c * NPAD + r0, RPT)])


def _dinv_from_hist(hist):
    deg = jnp.sum(hist, axis=0)[0:N_NODES, None] + 1.0
    return lax.rsqrt(deg)


def _tc1_body(doc_ref, word_ref, linw_ref, linb_ref, w1_ref, hist_ref, g1_ref):
    dinv = _dinv_from_hist(hist_ref[...])
    wf = jnp.dot(word_ref[...], linw_ref[...],
                 preferred_element_type=jnp.float32) + linb_ref[...]
    hd = jnp.dot(doc_ref[...], w1_ref[...], preferred_element_type=jnp.float32)
    hw = jnp.dot(wf, w1_ref[...], preferred_element_type=jnp.float32)
    g1_ref[0:N_DOC, :] = dinv[0:N_DOC] * hd
    g1_ref[N_DOC:N_NODES, :] = dinv[N_DOC:N_NODES] * hw


def _tc2_body(hist_ref, acc_ref, g1_ref, b1_ref, w2_ref, g2_ref):
    dinv = _dinv_from_hist(hist_ref[...])
    t = acc_ref[0:N_NODES, :] + acc_ref[NPAD:NPAD + N_NODES, :] + g1_ref[...]
    z = jnp.maximum(dinv * t + b1_ref[...], 0.0)
    g2_ref[...] = dinv * jnp.dot(z, w2_ref[...],
                                 preferred_element_type=jnp.float32)


def _tc3_body(hist_ref, acc_ref, g2_ref, b2_ref, out_ref):
    dinv = _dinv_from_hist(hist_ref[...])
    t = acc_ref[0:N_NODES, :] + acc_ref[NPAD:NPAD + N_NODES, :] + g2_ref[...]
    out_ref[...] = dinv * t + b2_ref[...]


def _vmem_call(body, n_in, out_shape):
    return pl.pallas_call(
        body,
        out_shape=out_shape,
        in_specs=[pl.BlockSpec(memory_space=pltpu.VMEM)] * n_in,
        out_specs=pl.BlockSpec(memory_space=pltpu.VMEM),
    )


def kernel(doc_features, word_features, edge_index, mode,
           lin_W, lin_b, W1, b1, W2, b2):
    src = edge_index[0].astype(jnp.int32)
    dst = edge_index[1].astype(jnp.int32)

    zerosd = jnp.zeros((NPAD, D), jnp.float32)

    deg_k, scat_k = _sc_kernels()
    hist = deg_k(dst)

    g1 = _vmem_call(_tc1_body, 6,
                    jax.ShapeDtypeStruct((N_NODES, D), jnp.float32))(
        doc_features, word_features, lin_W, lin_b.reshape(1, D), W1, hist)

    acc1 = scat_k(src, dst, g1, zerosd)

    g2 = _vmem_call(_tc2_body, 5,
                    jax.ShapeDtypeStruct((N_NODES, D), jnp.float32))(
        hist, acc1, g1, b1.reshape(1, D), W2)

    acc2 = scat_k(src, dst, g2, zerosd)

    out = _vmem_call(_tc3_body, 4,
                     jax.ShapeDtypeStruct((N_NODES, D), jnp.float32))(
        hist, acc2, g2, b2.reshape(1, D))
    return out


# speedup vs baseline: 34.0630x; 1.1769x over previous
import functools

import jax
import jax.numpy as jnp
from jax import lax
from jax.experimental import pallas as pl
from jax.experimental.pallas import tpu as pltpu
from jax.experimental.pallas import tpu_sc as plsc

N_DOC = 2000
N_NODES = 10000
N_EDGES = 320000
D = 128
NPAD = 10240

NC, NS = 2, 16
NW = NC * NS
EPT = N_EDGES // NW
K = 80
NSTEPS = EPT // K
RPT = NPAD // NS
L = 16
CH = 2000
NBUF = 4


@functools.cache
def _sc_kernels():
    mesh = plsc.VectorSubcoreMesh(core_axis_name="c", subcore_axis_name="s",
                                  num_cores=NC, num_subcores=NS)
    deg = functools.partial(
        pl.kernel,
        out_type=jax.ShapeDtypeStruct((NW, NPAD), jnp.float32),
        mesh=mesh,
        compiler_params=pltpu.CompilerParams(needs_layout_passes=False),
        scratch_types=[
            pltpu.VMEM((CH,), jnp.int32),
            pltpu.VMEM((NPAD,), jnp.float32),
        ],
    )(_deg_body)
    scat = functools.partial(
        pl.kernel,
        out_type=jax.ShapeDtypeStruct((NC * NPAD, D), jnp.float32),
        mesh=mesh,
        scratch_types=(
            [pltpu.VMEM((K,), jnp.int32)] * NBUF
            + [pltpu.VMEM((K,), jnp.int32)] * NBUF
            + [pltpu.VMEM((K, D), jnp.float32)] * NBUF
            + [pltpu.VMEM_SHARED((NPAD, D), jnp.float32)]
            + [pltpu.SemaphoreType.DMA] * (3 * NBUF)
        ),
    )(_edge_scatter_body)
    return deg, scat


def _deg_body(dst_hbm, out_hbm, didx_v, hist_v):
    c = lax.axis_index("c")
    s = lax.axis_index("s")
    wid = c * NS + s
    base = wid * EPT
    zero = jnp.zeros((L,), jnp.float32)

    @pl.loop(0, NPAD // L)
    def _(i):
        hist_v[pl.ds(i * L, L)] = zero

    ones = jnp.ones((L,), jnp.float32)

    @pl.loop(0, EPT // CH)
    def _(jc):
        pltpu.sync_copy(dst_hbm.at[pl.ds(base + jc * CH, CH)], didx_v)

        @pl.loop(0, CH // L)
        def _(i):
            idx = didx_v[pl.ds(i * L, L)]
            plsc.addupdate_scatter(hist_v, [idx], ones)

    pltpu.sync_copy(hist_v, out_hbm.at[wid])


def _edge_scatter_body(src_hbm, dst_hbm, g_hbm, zerosd_hbm, out_hbm, *refs):
    sidx = refs[0:NBUF]
    didx = refs[NBUF:2 * NBUF]
    rows = refs[2 * NBUF:3 * NBUF]
    acc_s = refs[3 * NBUF]
    isem = refs[3 * NBUF + 1:4 * NBUF + 1]
    gsem = refs[4 * NBUF + 1:5 * NBUF + 1]
    ssem = refs[5 * NBUF + 1:6 * NBUF + 1]
    c = lax.axis_index("c")
    s = lax.axis_index("s")
    base = (c * NS + s) * EPT
    r0 = s * RPT

    pltpu.sync_copy(zerosd_hbm.at[pl.ds(r0, RPT)], acc_s.at[pl.ds(r0, RPT)])
    plsc.subcore_barrier()

    def ifire(j, b):
        e0 = base + j * K
        pltpu.async_copy(src_hbm.at[pl.ds(e0, K)], sidx[b], isem[b])
        pltpu.async_copy(dst_hbm.at[pl.ds(e0, K)], didx[b], isem[b])

    def iwait(b):
        pltpu.make_async_copy(src_hbm.at[pl.ds(base, K)], sidx[b],
                              isem[b]).wait()
        pltpu.make_async_copy(dst_hbm.at[pl.ds(base, K)], didx[b],
                              isem[b]).wait()

    def gfire(b):
        pltpu.async_copy(g_hbm.at[sidx[b]], rows[b], gsem[b])

    def gwait(b):
        pltpu.make_async_copy(g_hbm.at[sidx[b]], rows[b], gsem[b]).wait()

    def sfire(b):
        pltpu.async_copy(rows[b], acc_s.at[didx[b]], ssem[b], add=True)

    def swait(b):
        pltpu.make_async_copy(rows[b], acc_s.at[didx[b]], ssem[b]).wait()

    def body(j, m, do_swait=True, do_i=True, do_g=True):
        if do_swait:
            swait((m + NBUF - 1) % NBUF)
        if do_i:
            ifire(j + 3, (m + 3) % NBUF)
        if do_g:
            iwait((m + 2) % NBUF)
            gfire((m + 2) % NBUF)
        gwait(m)
        sfire(m)

    ifire(0, 0)
    ifire(1, 1)
    ifire(2, 2)
    iwait(0)
    gfire(0)
    iwait(1)
    gfire(1)
    body(0, 0, do_swait=False)
    body(1, 1)
    body(2, 2)

    NGRP = (NSTEPS - 3 - 6) // NBUF

    @pl.loop(0, NGRP)
    def _(p):
        j = 3 + NBUF * p
        for t in range(NBUF):
            body(j + t, (3 + t) % NBUF)

    for j in range(3 + NBUF * NGRP, NSTEPS):
        body(j, j % NBUF, do_i=(j + 3 < NSTEPS), do_g=(j + 2 < NSTEPS))
    swait((NSTEPS - 1) % NBUF)

    plsc.subcore_barrier()
    pltpu.sync_copy(acc_s.at[pl.ds(r0, RPT)],
                    out_hbm.at[pl.ds(c * NPAD + r0, RPT)])


def _dinv_from_hist(hist):
    deg = jnp.sum(hist, axis=0)[0:N_NODES, None] + 1.0
    return lax.rsqrt(deg)


def _tc1_body(doc_ref, word_ref, linw_ref, linb_ref, w1_ref, hist_ref, g1_ref):
    dinv = _dinv_from_hist(hist_ref[...])
    wf = jnp.dot(word_ref[...], linw_ref[...],
                 preferred_element_type=jnp.float32) + linb_ref[...]
    hd = jnp.dot(doc_ref[...], w1_ref[...], preferred_element_type=jnp.float32)
    hw = jnp.dot(wf, w1_ref[...], preferred_element_type=jnp.float32)
    g1_ref[0:N_DOC, :] = dinv[0:N_DOC] * hd
    g1_ref[N_DOC:N_NODES, :] = dinv[N_DOC:N_NODES] * hw


def _tc2_body(hist_ref, acc_ref, g1_ref, b1_ref, w2_ref, g2_ref):
    dinv = _dinv_from_hist(hist_ref[...])
    t = acc_ref[0:N_NODES, :] + acc_ref[NPAD:NPAD + N_NODES, :] + g1_ref[...]
    z = jnp.maximum(dinv * t + b1_ref[...], 0.0)
    g2_ref[...] = dinv * jnp.dot(z, w2_ref[...],
                                 preferred_element_type=jnp.float32)


def _tc3_body(hist_ref, acc_ref, g2_ref, b2_ref, out_ref):
    dinv = _dinv_from_hist(hist_ref[...])
    t = acc_ref[0:N_NODES, :] + acc_ref[NPAD:NPAD + N_NODES, :] + g2_ref[...]
    out_ref[...] = dinv * t + b2_ref[...]


def _vmem_call(body, n_in, out_shape):
    return pl.pallas_call(
        body,
        out_shape=out_shape,
        in_specs=[pl.BlockSpec(memory_space=pltpu.VMEM)] * n_in,
        out_specs=pl.BlockSpec(memory_space=pltpu.VMEM),
    )


def kernel(doc_features, word_features, edge_index, mode,
           lin_W, lin_b, W1, b1, W2, b2):
    src = edge_index[0].astype(jnp.int32)
    dst = edge_index[1].astype(jnp.int32)

    zerosd = jnp.zeros((NPAD, D), jnp.float32)

    deg_k, scat_k = _sc_kernels()
    hist = deg_k(dst)

    g1 = _vmem_call(_tc1_body, 6,
                    jax.ShapeDtypeStruct((N_NODES, D), jnp.float32))(
        doc_features, word_features, lin_W, lin_b.reshape(1, D), W1, hist)

    acc1 = scat_k(src, dst, g1, zerosd)

    g2 = _vmem_call(_tc2_body, 5,
                    jax.ShapeDtypeStruct((N_NODES, D), jnp.float32))(
        hist, acc1, g1, b1.reshape(1, D), W2)

    acc2 = scat_k(src, dst, g2, zerosd)

    out = _vmem_call(_tc3_body, 4,
                     jax.ShapeDtypeStruct((N_NODES, D), jnp.float32))(
        hist, acc2, g2, b2.reshape(1, D))
    return out
